# baseline (speedup 1.0000x reference)
import functools

import jax
import jax.numpy as jnp
from jax import lax
from jax.experimental import pallas as pl
from jax.experimental.pallas import tpu as pltpu
from jax.experimental.pallas import tpu_sc as plsc

NN = 10000
EE = 160000
DD = 256
LANE = 16
NCORE = 2
NSUB = 16
NW = NCORE * NSUB
SEG = 320
NPAD = NW * SEG
CHA = 2000
NCH = EE // CHA
CAPC = 2048
STG = CAPC + 64
GB = 64
SENT = -3.0e38

_mesh = plsc.VectorSubcoreMesh(core_axis_name="c", subcore_axis_name="s")


def _worker_id():
    return lax.axis_index("s") * NCORE + lax.axis_index("c")


def _phase_a(src, dst):
    @functools.partial(
        pl.kernel,
        mesh=_mesh,
        out_type=(
            jax.ShapeDtypeStruct((NW, NCH, CAPC), jnp.int32),
            jax.ShapeDtypeStruct((NW, NCH, LANE), jnp.int32),
        ),
        compiler_params=pltpu.CompilerParams(needs_layout_passes=False),
        scratch_types=[
            pltpu.VMEM((STG,), jnp.int32),
            pltpu.VMEM((CHA,), jnp.int32),
            pltpu.VMEM((CHA,), jnp.int32),
            pltpu.VMEM((NCH, LANE), jnp.int32),
            pltpu.SemaphoreType.DMA,
        ],
    )
    def k(src_h, dst_h, list_h, mcnt_h, staging, sbuf, dbuf, mcbuf, sem):
        w = _worker_id()
        lo_v = jnp.zeros((LANE,), jnp.int32) + w * SEG
        hi_v = lo_v + SEG
        zz = jnp.zeros((LANE,), jnp.int32)

        def chunk_body(ci, carry):
            cs = pltpu.async_copy(src_h.at[pl.ds(ci * CHA, CHA)], sbuf, sem)
            cd = pltpu.async_copy(dst_h.at[pl.ds(ci * CHA, CHA)], dbuf, sem)
            cd.wait()
            cs.wait()

            def scan_body(k16, ptr):
                d = dbuf[pl.ds(k16 * LANE, LANE)]
                s = sbuf[pl.ds(k16 * LANE, LANE)]
                m = jnp.logical_and(d >= lo_v, d < hi_v)
                pack = s * 512 + (d - lo_v)
                mi = jnp.where(m, 1, 0).astype(jnp.int32)
                idx = (plsc.cumsum(mi) - mi) + (zz + ptr)
                plsc.store_scatter(staging, [idx], pack, mask=m)
                pc = plsc.all_reduce_population_count(m)
                return ptr + pc[0]

            ptr = lax.fori_loop(0, CHA // LANE, scan_body, 0)
            pad = zz + SEG
            for g in range(4):
                staging[pl.ds(ptr + g * LANE, LANE)] = pad
            mcbuf[ci, :] = zz + ptr
            pltpu.async_copy(staging.at[pl.ds(0, CAPC)], list_h.at[w, ci],
                             sem).wait()
            return carry

        lax.fori_loop(0, NCH, chunk_body, 0)
        pltpu.async_copy(mcbuf, mcnt_h.at[w], sem).wait()

    return k(src, dst)


def _pool(feat, lists, mcnts):
    @functools.partial(
        pl.kernel,
        mesh=_mesh,
        out_type=jax.ShapeDtypeStruct((NPAD, DD), jnp.float32),
        compiler_params=pltpu.CompilerParams(needs_layout_passes=False),
        scratch_types=[
            pltpu.VMEM((SEG + 1, DD), jnp.float32),
            pltpu.VMEM((GB, DD), jnp.float32),
            pltpu.VMEM((GB,), jnp.int32),
            pltpu.VMEM((GB,), jnp.int32),
            pltpu.VMEM((NCH, LANE), jnp.int32),
            pltpu.SemaphoreType.DMA,
        ],
    )
    def k(feat_h, list_h, mcnt_h, out_h, acc, rowbuf, lbuf, idxbuf, mcbuf, sem):
        w = _worker_id()
        sent = jnp.full((LANE,), SENT, jnp.float32)
        nine = jnp.full((LANE,), 9, jnp.int32)

        def init_body(r, _):
            for c in range(DD // LANE):
                acc[r, pl.ds(c * LANE, LANE)] = sent
            return 0

        lax.fori_loop(0, SEG, init_body, 0)
        pltpu.async_copy(mcnt_h.at[w], mcbuf, sem).wait()

        mask511 = jnp.full((LANE,), 511, jnp.int32)

        def ci_body(ci, _):
            mc = mcbuf[ci, :][0]
            nb = (mc + (GB - 1)) // GB

            def b_body(b, __):
                pltpu.async_copy(list_h.at[w, ci, pl.ds(b * GB, GB)], lbuf,
                                 sem).wait()
                for g in range(GB // LANE):
                    sl = pl.ds(g * LANE, LANE)
                    idxbuf[sl] = lax.shift_right_logical(lbuf[sl], nine)
                pltpu.async_copy(feat_h.at[idxbuf], rowbuf, sem).wait()

                def g_body(g, ___):
                    pks = lbuf[pl.ds(g * LANE, LANE)]
                    dls = jnp.bitwise_and(pks, mask511)
                    for j in range(LANE):
                        dl = dls[j]
                        r = g * LANE + j
                        for c in range(DD // LANE):
                            sl = pl.ds(c * LANE, LANE)
                            acc[dl, sl] = jnp.maximum(acc[dl, sl],
                                                      rowbuf[r, sl])
                    return 0

                lax.fori_loop(0, GB // LANE, g_body, 0)
                return 0

            lax.fori_loop(0, nb, b_body, 0)
            return 0

        lax.fori_loop(0, NCH, ci_body, 0)
        pltpu.async_copy(acc.at[pl.ds(0, SEG)],
                         out_h.at[pl.ds(w * SEG, SEG)], sem).wait()

    return k(feat, lists, mcnts)


_ROWS = 1000


def _tc_linear_relu(x, W, b):
    def body(x_ref, w_ref, b_ref, o_ref):
        o_ref[...] = jnp.maximum(
            jnp.dot(x_ref[...], w_ref[...],
                    preferred_element_type=jnp.float32) + b_ref[...], 0.0)

    return pl.pallas_call(
        body,
        grid=(NN // _ROWS,),
        in_specs=[
            pl.BlockSpec((_ROWS, DD), lambda i: (i, 0)),
            pl.BlockSpec((DD, DD), lambda i: (0, 0)),
            pl.BlockSpec((1, DD), lambda i: (0, 0)),
        ],
        out_specs=pl.BlockSpec((_ROWS, DD), lambda i: (i, 0)),
        out_shape=jax.ShapeDtypeStruct((NN, DD), jnp.float32),
    )(x, W, b)


def _tc_merge_relu(x, p, Wa, Wb, b):
    def body(x_ref, p_ref, wa_ref, wb_ref, b_ref, o_ref):
        pp = jnp.where(p_ref[...] < -1.0e38, 0.0, p_ref[...])
        t = (jnp.dot(x_ref[...], wa_ref[...], preferred_element_type=jnp.float32)
             + jnp.dot(pp, wb_ref[...], preferred_element_type=jnp.float32)
             + b_ref[...])
        o_ref[...] = jnp.maximum(t, 0.0)

    return pl.pallas_call(
        body,
        grid=(NN // _ROWS,),
        in_specs=[
            pl.BlockSpec((_ROWS, DD), lambda i: (i, 0)),
            pl.BlockSpec((_ROWS, DD), lambda i: (i, 0)),
            pl.BlockSpec((DD, DD), lambda i: (0, 0)),
            pl.BlockSpec((DD, DD), lambda i: (0, 0)),
            pl.BlockSpec((1, DD), lambda i: (0, 0)),
        ],
        out_specs=pl.BlockSpec((_ROWS, DD), lambda i: (i, 0)),
        out_shape=jax.ShapeDtypeStruct((NN, DD), jnp.float32),
    )(x, p, Wa, Wb, b)


def _tc_merge_ln(h, p, Wa, Wb, b, g, bb):
    def body(h_ref, p_ref, wa_ref, wb_ref, b_ref, g_ref, bb_ref, o_ref):
        hh = h_ref[...]
        pp = jnp.where(p_ref[...] < -1.0e38, 0.0, p_ref[...])
        t = (jnp.dot(hh, wa_ref[...], preferred_element_type=jnp.float32)
             + jnp.dot(pp, wb_ref[...], preferred_element_type=jnp.float32)
             + b_ref[...])
        u = jnp.maximum(t, 0.0) + hh
        mu = jnp.mean(u, axis=-1, keepdims=True)
        dm = u - mu
        var = jnp.mean(dm * dm, axis=-1, keepdims=True)
        o_ref[...] = dm * lax.rsqrt(var + 1e-5) * g_ref[...] + bb_ref[...]

    return pl.pallas_call(
        body,
        grid=(NN // _ROWS,),
        in_specs=[
            pl.BlockSpec((_ROWS, DD), lambda i: (i, 0)),
            pl.BlockSpec((_ROWS, DD), lambda i: (i, 0)),
            pl.BlockSpec((DD, DD), lambda i: (0, 0)),
            pl.BlockSpec((DD, DD), lambda i: (0, 0)),
            pl.BlockSpec((1, DD), lambda i: (0, 0)),
            pl.BlockSpec((1, DD), lambda i: (0, 0)),
            pl.BlockSpec((1, DD), lambda i: (0, 0)),
        ],
        out_specs=pl.BlockSpec((_ROWS, DD), lambda i: (i, 0)),
        out_shape=jax.ShapeDtypeStruct((NN, DD), jnp.float32),
    )(h, p, Wa, Wb, b, g, bb)


def kernel(x, edge_index, W_pool1, b_pool1, W_merge1, b_merge1,
           W_merge2, b_merge2, ln2_g, ln2_b,
           W_merge3, b_merge3, ln3_g, ln3_b,
           W_merge4, b_merge4, ln4_g, ln4_b):
    src = edge_index[0]
    dst = edge_index[1]
    lists, mcnts = _phase_a(src, dst)

    ne = _tc_linear_relu(x, W_pool1, b_pool1.reshape(1, -1))
    pooled = _pool(ne, lists, mcnts)[:NN]
    h = _tc_merge_relu(x, pooled, W_merge1[:DD], W_merge1[DD:],
                       b_merge1.reshape(1, -1))
    for Wm, bm, g, bb in ((W_merge2, b_merge2, ln2_g, ln2_b),
                          (W_merge3, b_merge3, ln3_g, ln3_b),
                          (W_merge4, b_merge4, ln4_g, ln4_b)):
        pooled = _pool(h, lists, mcnts)[:NN]
        h = _tc_merge_ln(h, pooled, Wm[:DD], Wm[DD:], bm.reshape(1, -1),
                         g.reshape(1, -1), bb.reshape(1, -1))
    return h


# speedup vs baseline: 6.3316x; 6.3316x over previous
import functools

import jax
import jax.numpy as jnp
from jax import lax
from jax.experimental import pallas as pl
from jax.experimental.pallas import tpu as pltpu
from jax.experimental.pallas import tpu_sc as plsc

NN = 10000
EE = 160000
DD = 256
LANE = 16
NCORE = 2
NSUB = 16
NW = NCORE * NSUB
SEG = 320
NPAD = NW * SEG
CHA = 2000
NCH = EE // CHA
STGW = 4096
SMSK = STGW - 1
BLK = 2048
CAPL = EE + 2 * BLK
GB = 64
SENT = -3.0e38

_mesh = plsc.VectorSubcoreMesh(core_axis_name="c", subcore_axis_name="s")
_SC_PARAMS = pltpu.CompilerParams(needs_layout_passes=False)


def _worker_id():
    return lax.axis_index("s") * NCORE + lax.axis_index("c")


def _al(i, n=8):
    return pl.multiple_of(i, n)


def _phase_a(src, dst):
    @functools.partial(
        pl.kernel,
        mesh=_mesh,
        out_type=(
            jax.ShapeDtypeStruct((NW, CAPL), jnp.int32),
            jax.ShapeDtypeStruct((NW, LANE), jnp.int32),
        ),
        compiler_params=_SC_PARAMS,
        scratch_types=[
            pltpu.VMEM((STGW,), jnp.int32),
            pltpu.VMEM((CHA,), jnp.int32),
            pltpu.VMEM((CHA,), jnp.int32),
            pltpu.VMEM((CHA,), jnp.int32),
            pltpu.VMEM((CHA,), jnp.int32),
            pltpu.VMEM((LANE,), jnp.int32),
            pltpu.SemaphoreType.DMA,
            pltpu.SemaphoreType.DMA,
        ],
    )
    def k(src_h, dst_h, list_h, cnt_h, staging, sb0, sb1, db0, db1, cbuf,
          semc0, semc1):
        w = _worker_id()
        lo_v = jnp.zeros((LANE,), jnp.int32) + w * SEG
        hi_v = lo_v + SEG
        zz = jnp.zeros((LANE,), jnp.int32)
        wrap = jnp.full((LANE,), SMSK, jnp.int32)
        sbufs = (sb0, sb1)
        dbufs = (db0, db1)
        semcs = (semc0, semc1)

        def issue(ci, p):
            pltpu.async_copy(src_h.at[pl.ds(_al(ci * CHA), CHA)], sbufs[p],
                             semcs[p])
            pltpu.async_copy(dst_h.at[pl.ds(_al(ci * CHA), CHA)], dbufs[p],
                             semcs[p])

        def wait_chunk(ci, p):
            pltpu.make_async_copy(src_h.at[pl.ds(_al(ci * CHA), CHA)], sbufs[p],
                                  semcs[p]).wait()
            pltpu.make_async_copy(dst_h.at[pl.ds(_al(ci * CHA), CHA)], dbufs[p],
                                  semcs[p]).wait()

        issue(0, 0)
        issue(1, 1)

        def scan_chunk(p, ptr):
            def scan_body(k16, q):
                d = dbufs[p][pl.ds(k16 * LANE, LANE)]
                s = sbufs[p][pl.ds(k16 * LANE, LANE)]
                m = jnp.logical_and(d >= lo_v, d < hi_v)
                pack = s * 512 + (d - lo_v)
                mi = jnp.where(m, 1, 0).astype(jnp.int32)
                idx = jnp.bitwise_and((plsc.cumsum(mi) - mi) + (zz + q), wrap)
                plsc.store_scatter(staging, [idx], pack, mask=m)
                pc = plsc.all_reduce_population_count(m)
                return q + pc[0]

            return lax.fori_loop(0, CHA // LANE, scan_body, ptr)

        def maybe_flush(ptr_prev, ptr):
            fl_old = (ptr_prev // BLK) * BLK
            fl_new = (ptr // BLK) * BLK

            @pl.when(fl_new > fl_old)
            def _():
                fo = jnp.bitwise_and(fl_old, SMSK)
                pltpu.sync_copy(staging.at[pl.ds(_al(fo, BLK), BLK)],
                                list_h.at[w, pl.ds(_al(fl_old, BLK), BLK)])

        def pair_body(cp, ptr):
            p1 = ptr
            for p in range(2):
                ci = 2 * cp + p
                wait_chunk(ci, p)
                p2 = scan_chunk(p, p1)

                @pl.when(ci + 2 < NCH)
                def _():
                    issue(ci + 2, p)

                maybe_flush(p1, p2)
                p1 = p2
            return p1

        ptr = lax.fori_loop(0, NCH // 2, pair_body, 0)

        padv = zz + SEG
        for g in range(2 * GB // LANE):
            idxp = jnp.bitwise_and(zz + ptr + g * LANE
                                   + lax.iota(jnp.int32, LANE), wrap)
            plsc.store_scatter(staging, [idxp], padv)
        fl = (ptr // BLK) * BLK
        fo = jnp.bitwise_and(fl, SMSK)
        pltpu.sync_copy(staging.at[pl.ds(_al(fo, BLK), BLK)],
                        list_h.at[w, pl.ds(_al(fl, BLK), BLK)])
        fo2 = jnp.bitwise_xor(fo, BLK)
        pltpu.sync_copy(staging.at[pl.ds(_al(fo2, BLK), BLK)],
                        list_h.at[w, pl.ds(_al(fl + BLK, BLK), BLK)])
        cbuf[...] = zz + ptr
        pltpu.sync_copy(cbuf, cnt_h.at[w])

    return k(src, dst)


def _pool(feat, lists, cnts):
    @functools.partial(
        pl.kernel,
        mesh=_mesh,
        out_type=jax.ShapeDtypeStruct((NPAD, DD), jnp.float32),
        compiler_params=_SC_PARAMS,
        scratch_types=[
            pltpu.VMEM((SEG + 1, DD), jnp.float32),
            pltpu.VMEM((GB, DD), jnp.float32),
            pltpu.VMEM((GB, DD), jnp.float32),
            pltpu.VMEM((GB,), jnp.int32),
            pltpu.VMEM((GB,), jnp.int32),
            pltpu.VMEM((GB,), jnp.int32),
            pltpu.VMEM((GB,), jnp.int32),
            pltpu.VMEM((GB,), jnp.int32),
            pltpu.VMEM((GB,), jnp.int32),
            pltpu.VMEM((LANE,), jnp.int32),
            pltpu.SemaphoreType.DMA,
            pltpu.SemaphoreType.DMA,
            pltpu.SemaphoreType.DMA,
            pltpu.SemaphoreType.DMA,
        ],
    )
    def k(feat_h, list_h, cnt_h, out_h, acc, rb0, rb1, lb0, lb1, ib0, ib1,
          db0, db1, cbuf, seml0, seml1, semg0, semg1):
        w = _worker_id()
        sent = jnp.full((LANE,), SENT, jnp.float32)
        nine = jnp.full((LANE,), 9, jnp.int32)
        m511 = jnp.full((LANE,), 511, jnp.int32)
        rbufs = (rb0, rb1)
        lbufs = (lb0, lb1)
        ibufs = (ib0, ib1)
        dbufs = (db0, db1)
        semls = (seml0, seml1)
        semgs = (semg0, semg1)

        def init_body(r, _):
            for c in range(DD // LANE):
                acc[r, pl.ds(c * LANE, LANE)] = sent
            return 0

        lax.fori_loop(0, SEG, init_body, 0)

        pltpu.sync_copy(cnt_h.at[w], cbuf)
        cnt = cbuf[...][0]
        nb = (cnt + GB - 1) // GB
        nbp = (nb + 1) // 2
        nbexec = 2 * nbp

        def list_src(b):
            return list_h.at[w, pl.ds(_al(b * GB, GB), GB)]

        @pl.when(nbexec > 0)
        def _():
            pltpu.async_copy(list_src(0), lb0, seml0)
            pltpu.async_copy(list_src(1), lb1, seml1)

        def do_accum(q):
            def g_body(g, _):
                dls = dbufs[q][pl.ds(g * LANE, LANE)]
                for j in range(LANE):
                    dl = dls[j]
                    r = g * LANE + j
                    for c in range(DD // LANE):
                        sl = pl.ds(c * LANE, LANE)
                        acc[dl, sl] = jnp.maximum(acc[dl, sl],
                                                  rbufs[q][r, sl])
                return 0

            lax.fori_loop(0, GB // LANE, g_body, 0)

        def wait_gather(q):
            pltpu.make_async_copy(feat_h.at[pl.ds(0, GB)], rbufs[q],
                                  semgs[q]).wait()

        def pair_body(i, _):
            for p in range(2):
                b = 2 * i + p
                pltpu.make_async_copy(list_src(b), lbufs[p], semls[p]).wait()
                for g in range(GB // LANE):
                    sl = pl.ds(g * LANE, LANE)
                    pk = lbufs[p][sl]
                    ibufs[p][sl] = lax.shift_right_logical(pk, nine)
                    dbufs[p][sl] = jnp.bitwise_and(pk, m511)

                @pl.when(b + 2 < nbexec)
                def _():
                    pltpu.async_copy(list_src(b + 2), lbufs[p], semls[p])

                pltpu.async_copy(feat_h.at[ibufs[p]], rbufs[p], semgs[p])

                @pl.when(b >= 1)
                def _():
                    wait_gather(1 - p)
                    do_accum(1 - p)
            return 0

        lax.fori_loop(0, nbp, pair_body, 0)

        @pl.when(nbexec > 0)
        def _():
            wait_gather(1)
            do_accum(1)

        pltpu.sync_copy(acc.at[pl.ds(0, SEG)],
                        out_h.at[pl.ds(_al(w * SEG, SEG), SEG)])

    return k(feat, lists, cnts)


_ROWS = 1000


def _tc_linear_relu(x, W, b):
    def body(x_ref, w_ref, b_ref, o_ref):
        o_ref[...] = jnp.maximum(
            jnp.dot(x_ref[...], w_ref[...],
                    preferred_element_type=jnp.float32) + b_ref[...], 0.0)

    return pl.pallas_call(
        body,
        grid=(NN // _ROWS,),
        in_specs=[
            pl.BlockSpec((_ROWS, DD), lambda i: (i, 0)),
            pl.BlockSpec((DD, DD), lambda i: (0, 0)),
            pl.BlockSpec((1, DD), lambda i: (0, 0)),
        ],
        out_specs=pl.BlockSpec((_ROWS, DD), lambda i: (i, 0)),
        out_shape=jax.ShapeDtypeStruct((NN, DD), jnp.float32),
    )(x, W, b)


def _tc_merge_relu(x, p, Wa, Wb, b):
    def body(x_ref, p_ref, wa_ref, wb_ref, b_ref, o_ref):
        pp = jnp.where(p_ref[...] < -1.0e38, 0.0, p_ref[...])
        t = (jnp.dot(x_ref[...], wa_ref[...], preferred_element_type=jnp.float32)
             + jnp.dot(pp, wb_ref[...], preferred_element_type=jnp.float32)
             + b_ref[...])
        o_ref[...] = jnp.maximum(t, 0.0)

    return pl.pallas_call(
        body,
        grid=(NN // _ROWS,),
        in_specs=[
            pl.BlockSpec((_ROWS, DD), lambda i: (i, 0)),
            pl.BlockSpec((_ROWS, DD), lambda i: (i, 0)),
            pl.BlockSpec((DD, DD), lambda i: (0, 0)),
            pl.BlockSpec((DD, DD), lambda i: (0, 0)),
            pl.BlockSpec((1, DD), lambda i: (0, 0)),
        ],
        out_specs=pl.BlockSpec((_ROWS, DD), lambda i: (i, 0)),
        out_shape=jax.ShapeDtypeStruct((NN, DD), jnp.float32),
    )(x, p, Wa, Wb, b)


def _tc_merge_ln(h, p, Wa, Wb, b, g, bb):
    def body(h_ref, p_ref, wa_ref, wb_ref, b_ref, g_ref, bb_ref, o_ref):
        hh = h_ref[...]
        pp = jnp.where(p_ref[...] < -1.0e38, 0.0, p_ref[...])
        t = (jnp.dot(hh, wa_ref[...], preferred_element_type=jnp.float32)
             + jnp.dot(pp, wb_ref[...], preferred_element_type=jnp.float32)
             + b_ref[...])
        u = jnp.maximum(t, 0.0) + hh
        mu = jnp.mean(u, axis=-1, keepdims=True)
        dm = u - mu
        var = jnp.mean(dm * dm, axis=-1, keepdims=True)
        o_ref[...] = dm * lax.rsqrt(var + 1e-5) * g_ref[...] + bb_ref[...]

    return pl.pallas_call(
        body,
        grid=(NN // _ROWS,),
        in_specs=[
            pl.BlockSpec((_ROWS, DD), lambda i: (i, 0)),
            pl.BlockSpec((_ROWS, DD), lambda i: (i, 0)),
            pl.BlockSpec((DD, DD), lambda i: (0, 0)),
            pl.BlockSpec((DD, DD), lambda i: (0, 0)),
            pl.BlockSpec((1, DD), lambda i: (0, 0)),
            pl.BlockSpec((1, DD), lambda i: (0, 0)),
            pl.BlockSpec((1, DD), lambda i: (0, 0)),
        ],
        out_specs=pl.BlockSpec((_ROWS, DD), lambda i: (i, 0)),
        out_shape=jax.ShapeDtypeStruct((NN, DD), jnp.float32),
    )(h, p, Wa, Wb, b, g, bb)


def kernel(x, edge_index, W_pool1, b_pool1, W_merge1, b_merge1,
           W_merge2, b_merge2, ln2_g, ln2_b,
           W_merge3, b_merge3, ln3_g, ln3_b,
           W_merge4, b_merge4, ln4_g, ln4_b):
    src = edge_index[0]
    dst = edge_index[1]
    lists, cnts = _phase_a(src, dst)

    ne = _tc_linear_relu(x, W_pool1, b_pool1.reshape(1, -1))
    pooled = _pool(ne, lists, cnts)[:NN]
    h = _tc_merge_relu(x, pooled, W_merge1[:DD], W_merge1[DD:],
                       b_merge1.reshape(1, -1))
    for Wm, bm, g, bb in ((W_merge2, b_merge2, ln2_g, ln2_b),
                          (W_merge3, b_merge3, ln3_g, ln3_b),
                          (W_merge4, b_merge4, ln4_g, ln4_b)):
        pooled = _pool(h, lists, cnts)[:NN]
        h = _tc_merge_ln(h, pooled, Wm[:DD], Wm[DD:], bm.reshape(1, -1),
                         g.reshape(1, -1), bb.reshape(1, -1))
    return h
